# baseline (speedup 1.0000x reference)
import functools

import jax
import jax.numpy as jnp
from jax import lax
from jax.experimental import pallas as pl
from jax.experimental.pallas import tpu as pltpu
from jax.experimental.pallas import tpu_sc as plsc

B = 2
S = 2048
D = 1024
H = 16
DH = D // H
POOL = 512
TOPK = 128
NSEL = 5 * TOPK
ROWS = B * NSEL

_NC, _NS = 2, 16
_NW = _NC * _NS
_RPW = ROWS // _NW

F32 = jnp.float32
BF16 = jnp.bfloat16


def _sc_gather_body(table_hbm, idx_hbm, out_hbm, idx_v, rows_v, sem):
    wid = lax.axis_index("s") * _NC + lax.axis_index("c")
    base = wid * _RPW
    pltpu.sync_copy(idx_hbm.at[pl.ds(base, _RPW)], idx_v)
    pltpu.async_copy(table_hbm.at[idx_v], rows_v, sem).wait()
    pltpu.sync_copy(rows_v, out_hbm.at[pl.ds(base, _RPW)])


def _gather_rows(table, idx_flat):
    mesh = plsc.VectorSubcoreMesh(core_axis_name="c", subcore_axis_name="s")
    fn = functools.partial(
        pl.kernel,
        mesh=mesh,
        out_type=jax.ShapeDtypeStruct((ROWS, D), F32),
        scratch_types=[
            pltpu.VMEM((_RPW,), jnp.int32),
            pltpu.VMEM((_RPW, D), F32),
            pltpu.SemaphoreType.DMA,
        ],
    )(_sc_gather_body)
    return fn(table, idx_flat)


_QKV_BLK = 256
_Q_PRESCALE = 0.125 * 1.4426950408889634


def _qkv_body(x_ref, g_ref, sqk_ref, sv_ref, sq_ref, sk_ref, sv2_ref,
              q_ref, k_ref, v_ref):
    x = x_ref[0].astype(BF16)
    g = g_ref[0]
    a_down = g[0:TOPK]
    b_down = g[TOPK:2 * TOPK]
    rq = g[2 * TOPK:3 * TOPK]
    rk = g[3 * TOPK:4 * TOPK]
    rv = g[4 * TOPK:5 * TOPK]
    dn = (((1,), (1,)), ((), ()))
    up = (((1,), (0,)), ((), ()))
    h_qk = lax.dot_general(x, a_down, dn, preferred_element_type=F32)
    h_v = lax.dot_general(x, b_down, dn, preferred_element_type=F32)
    sqk = sqk_ref[0]
    hq = (h_qk * (sqk * sq_ref[0])).astype(BF16)
    hk = (h_qk * (sqk * sk_ref[0])).astype(BF16)
    hv = (h_v * (sv_ref[0] * sv2_ref[0])).astype(BF16)
    q_ref[0] = (lax.dot_general(hq, rq, up, preferred_element_type=F32)
                * _Q_PRESCALE).astype(BF16)
    k_ref[0] = lax.dot_general(hk, rk, up, preferred_element_type=F32
                               ).astype(BF16)
    v_ref[0] = lax.dot_general(hv, rv, up, preferred_element_type=F32
                               ).astype(BF16)


def _qkv_call(x, g, s_qk, s_v, s_q, s_k, s_v2):
    grid = (B, S // _QKV_BLK)
    soft_spec = pl.BlockSpec((1, 1, TOPK), lambda b, s: (b, 0, 0))
    out_spec = pl.BlockSpec((1, _QKV_BLK, D), lambda b, s: (b, s, 0))
    return pl.pallas_call(
        _qkv_body,
        grid=grid,
        in_specs=[
            pl.BlockSpec((1, _QKV_BLK, D), lambda b, s: (b, s, 0)),
            pl.BlockSpec((1, NSEL, D), lambda b, s: (b, 0, 0)),

            soft_spec, soft_spec, soft_spec, soft_spec, soft_spec,
        ],
        out_specs=[out_spec, out_spec, out_spec],
        out_shape=[jax.ShapeDtypeStruct((B, S, D), BF16)] * 3,
        compiler_params=pltpu.CompilerParams(
            dimension_semantics=("parallel", "parallel")),
    )(x, g, s_qk, s_v, s_q, s_k, s_v2)


_BQ = 512
_SM_SCALE = 1.0 / (DH ** 0.5)


def _flash_body(q_ref, k_ref, v_ref, o_ref):
    qi = pl.program_id(2)
    q = q_ref[0]
    q1 = q[:, :DH]
    q2 = q[:, DH:]
    neg = jnp.float32(-1e30)
    ones_col = jnp.ones((_BQ, 1), BF16)
    pv_dn = (((1,), (0,)), ((), ()))

    def scores(c):
        kc = k_ref[0, pl.ds(c * _BQ, _BQ), :]
        sdn = (((1,), (1,)), ((), ()))
        s1 = lax.dot_general(q1, kc[:, :DH], sdn, preferred_element_type=F32)
        s2 = lax.dot_general(q2, kc[:, DH:], sdn, preferred_element_type=F32)
        return s1, s2

    def one_head(s, vc, mask, m, l, acc):
        if mask is not None:
            s = jnp.where(mask, s, neg)
        m_new = jnp.maximum(m, jnp.max(s, axis=1, keepdims=True))
        p = jnp.exp2(s - m_new).astype(BF16)
        alpha = jnp.exp2(m - m_new)
        vc_aug = jnp.concatenate([vc, ones_col], axis=1)
        av = lax.dot_general(p, vc_aug, pv_dn, preferred_element_type=F32)
        l_new = l * alpha + av[:, DH:DH + 1]
        acc_new = acc * alpha + av[:, :DH]
        return m_new, l_new, acc_new

    def chunk(c, carry, s1, s2, mask):
        m1, l1, a1, m2, l2, a2 = carry
        vc = v_ref[0, pl.ds(c * _BQ, _BQ), :]
        m1, l1, a1 = one_head(s1, vc[:, :DH], mask, m1, l1, a1)
        m2, l2, a2 = one_head(s2, vc[:, DH:], mask, m2, l2, a2)
        return m1, l1, a1, m2, l2, a2

    minit = jnp.full((_BQ, 1), -jnp.inf, F32)
    linit = jnp.zeros((_BQ, 1), F32)
    ainit = jnp.zeros((_BQ, DH), F32)

    s1, s2 = scores(0)
    def body(c, carry):
        *st, s1c, s2c = carry
        s1n, s2n = scores(c + 1)
        st = chunk(c, tuple(st), s1c, s2c, None)
        return (*st, s1n, s2n)
    *carry, s1, s2 = lax.fori_loop(
        0, qi, body, (minit, linit, ainit, minit, linit, ainit, s1, s2))
    rows = lax.broadcasted_iota(jnp.int32, (_BQ, _BQ), 0)
    cols = lax.broadcasted_iota(jnp.int32, (_BQ, _BQ), 1)
    m1, l1, a1, m2, l2, a2 = chunk(qi, tuple(carry), s1, s2, cols <= rows)
    o_ref[0] = jnp.concatenate([a1 / l1, a2 / l2], axis=1).astype(BF16)


def _flash_call(q, k, v):
    grid = (B, H // 2, S // _BQ)
    kv_spec = pl.BlockSpec((1, S, 2 * DH), lambda b, hp, s: (b, 0, hp))
    q_spec = pl.BlockSpec((1, _BQ, 2 * DH), lambda b, hp, s: (b, s, hp))
    return pl.pallas_call(
        _flash_body,
        grid=grid,
        in_specs=[q_spec, kv_spec, kv_spec],
        out_specs=q_spec,
        out_shape=jax.ShapeDtypeStruct((B, S, D), BF16),
        compiler_params=pltpu.CompilerParams(
            dimension_semantics=("parallel", "parallel", "arbitrary")),
    )(q, k, v)


_PROJ_BLK = 512


def _proj_body(a_ref, w_ref, o_ref):
    o_ref[0] = lax.dot_general(a_ref[0], w_ref[...],
                               (((1,), (1,)), ((), ())),
                               preferred_element_type=F32)


def _proj_call(a, w):
    grid = (B, S // _PROJ_BLK)
    return pl.pallas_call(
        _proj_body,
        grid=grid,
        in_specs=[
            pl.BlockSpec((1, _PROJ_BLK, D), lambda b, s: (b, s, 0)),
            pl.BlockSpec((D, D), lambda b, s: (0, 0)),
        ],
        out_specs=pl.BlockSpec((1, _PROJ_BLK, D), lambda b, s: (b, s, 0)),
        out_shape=jax.ShapeDtypeStruct((B, S, D), F32),
        compiler_params=pltpu.CompilerParams(
            dimension_semantics=("parallel", "parallel")),
    )(a, w)


def kernel(x, idx_qk, idx_v, idx_q, idx_k, idx_v2,
           soft_qk, soft_v, soft_q, soft_k, soft_v2,
           feature_qk_neurons, feature_v_neurons, relational_neurons,
           value_neurons, W_O):
    table = jnp.concatenate(
        [feature_qk_neurons, feature_v_neurons, relational_neurons,
         value_neurons], axis=0)
    idx_all = jnp.concatenate(
        [idx_qk, idx_v + POOL, idx_q + 2 * POOL, idx_k + 2 * POOL,
         idx_v2 + 3 * POOL], axis=1).astype(jnp.int32)
    g = _gather_rows(table, idx_all.reshape(ROWS))
    g = g.reshape(B, NSEL, D).astype(BF16)
    q, k, v = _qkv_call(
        x, g,
        soft_qk.reshape(B, 1, TOPK), soft_v.reshape(B, 1, TOPK),
        soft_q.reshape(B, 1, TOPK), soft_k.reshape(B, 1, TOPK),
        soft_v2.reshape(B, 1, TOPK))
    attn = _flash_call(q, k, v)
    return _proj_call(attn, W_O.astype(BF16))


# speedup vs baseline: 1.2024x; 1.2024x over previous
import functools

import jax
import jax.numpy as jnp
from jax import lax
from jax.experimental import pallas as pl
from jax.experimental.pallas import tpu as pltpu
from jax.experimental.pallas import tpu_sc as plsc

B = 2
S = 2048
D = 1024
H = 16
DH = D // H
POOL = 512
TOPK = 128
NSEL = 5 * TOPK
ROWS = B * NSEL

_NC, _NS = 2, 16
_NW = _NC * _NS
_RPW = ROWS // _NW

F32 = jnp.float32
BF16 = jnp.bfloat16


def _sc_gather_body(table_hbm, idx_hbm, out_hbm, idx_v, rows_v, sem):
    wid = lax.axis_index("s") * _NC + lax.axis_index("c")
    base = wid * _RPW
    pltpu.sync_copy(idx_hbm.at[pl.ds(base, _RPW)], idx_v)
    pltpu.async_copy(table_hbm.at[idx_v], rows_v, sem).wait()
    pltpu.sync_copy(rows_v, out_hbm.at[pl.ds(base, _RPW)])


def _gather_rows(table, idx_flat):
    mesh = plsc.VectorSubcoreMesh(core_axis_name="c", subcore_axis_name="s")
    fn = functools.partial(
        pl.kernel,
        mesh=mesh,
        out_type=jax.ShapeDtypeStruct((ROWS, D), F32),
        scratch_types=[
            pltpu.VMEM((_RPW,), jnp.int32),
            pltpu.VMEM((_RPW, D), F32),
            pltpu.SemaphoreType.DMA,
        ],
    )(_sc_gather_body)
    return fn(table, idx_flat)


_QKV_BLK = 256
_Q_PRESCALE = 0.125 * 1.4426950408889634


def _qkv_body(x_ref, g_ref, sqk_ref, sv_ref, sq_ref, sk_ref, sv2_ref,
              q_ref, k_ref, v_ref):
    x = x_ref[0].astype(BF16)
    g = g_ref[0]
    a_down = g[0:TOPK]
    b_down = g[TOPK:2 * TOPK]
    rq = g[2 * TOPK:3 * TOPK]
    rk = g[3 * TOPK:4 * TOPK]
    rv = g[4 * TOPK:5 * TOPK]
    dn = (((1,), (1,)), ((), ()))
    up = (((1,), (0,)), ((), ()))
    h_qk = lax.dot_general(x, a_down, dn, preferred_element_type=F32)
    h_v = lax.dot_general(x, b_down, dn, preferred_element_type=F32)
    sqk = sqk_ref[0]
    hq = (h_qk * (sqk * sq_ref[0])).astype(BF16)
    hk = (h_qk * (sqk * sk_ref[0])).astype(BF16)
    hv = (h_v * (sv_ref[0] * sv2_ref[0])).astype(BF16)
    q_ref[0] = (lax.dot_general(hq, rq, up, preferred_element_type=F32)
                * _Q_PRESCALE).astype(BF16)
    k_ref[0] = lax.dot_general(hk, rk, up, preferred_element_type=F32
                               ).astype(BF16)
    v_ref[0] = lax.dot_general(hv, rv, up, preferred_element_type=F32
                               ).astype(BF16)


def _qkv_call(x, g, s_qk, s_v, s_q, s_k, s_v2):
    grid = (B, S // _QKV_BLK)
    soft_spec = pl.BlockSpec((1, 1, TOPK), lambda b, s: (b, 0, 0))
    out_spec = pl.BlockSpec((1, _QKV_BLK, D), lambda b, s: (b, s, 0))
    return pl.pallas_call(
        _qkv_body,
        grid=grid,
        in_specs=[
            pl.BlockSpec((1, _QKV_BLK, D), lambda b, s: (b, s, 0)),
            pl.BlockSpec((1, NSEL, D), lambda b, s: (b, 0, 0)),

            soft_spec, soft_spec, soft_spec, soft_spec, soft_spec,
        ],
        out_specs=[out_spec, out_spec, out_spec],
        out_shape=[jax.ShapeDtypeStruct((B, S, D), BF16)] * 3,
        compiler_params=pltpu.CompilerParams(
            dimension_semantics=("parallel", "parallel")),
    )(x, g, s_qk, s_v, s_q, s_k, s_v2)


_BQ = 512
_SM_SCALE = 1.0 / (DH ** 0.5)


def _flash_body(q_ref, k_ref, v_ref, o_ref):
    qi = pl.program_id(2)
    q = q_ref[0]
    q1 = q[:, :DH]
    q2 = q[:, DH:]
    neg = jnp.float32(-1e30)
    ones_col = jnp.ones((_BQ, 1), BF16)
    pv_dn = (((1,), (0,)), ((), ()))

    def one_head(s, vc, mask, m, l, acc):
        if mask is not None:
            s = jnp.where(mask, s, neg)
        m_new = jnp.maximum(m, jnp.max(s, axis=1, keepdims=True))
        p = jnp.exp2(s - m_new).astype(BF16)
        alpha = jnp.exp2(m - m_new)
        vc_aug = jnp.concatenate([vc, ones_col], axis=1)
        av = lax.dot_general(p, vc_aug, pv_dn, preferred_element_type=F32)
        l_new = l * alpha + av[:, DH:DH + 1]
        acc_new = acc * alpha + av[:, :DH]
        return m_new, l_new, acc_new

    def chunk(c, carry, mask):
        m1, l1, a1, m2, l2, a2 = carry
        kc = k_ref[0, pl.ds(c * _BQ, _BQ), :]
        vc = v_ref[0, pl.ds(c * _BQ, _BQ), :]
        sdn = (((1,), (1,)), ((), ()))
        s1 = lax.dot_general(q1, kc[:, :DH], sdn, preferred_element_type=F32)
        s2 = lax.dot_general(q2, kc[:, DH:], sdn, preferred_element_type=F32)
        m1, l1, a1 = one_head(s1, vc[:, :DH], mask, m1, l1, a1)
        m2, l2, a2 = one_head(s2, vc[:, DH:], mask, m2, l2, a2)
        return m1, l1, a1, m2, l2, a2

    minit = jnp.full((_BQ, 1), -jnp.inf, F32)
    linit = jnp.zeros((_BQ, 1), F32)
    ainit = jnp.zeros((_BQ, DH), F32)
    carry = lax.fori_loop(0, qi, lambda c, cr: chunk(c, cr, None),
                          (minit, linit, ainit, minit, linit, ainit))
    rows = lax.broadcasted_iota(jnp.int32, (_BQ, _BQ), 0)
    cols = lax.broadcasted_iota(jnp.int32, (_BQ, _BQ), 1)
    m1, l1, a1, m2, l2, a2 = chunk(qi, carry, cols <= rows)
    o_ref[0] = jnp.concatenate([a1 / l1, a2 / l2], axis=1).astype(BF16)


def _flash_call(q, k, v):
    grid = (B, H // 2, S // _BQ)
    kv_spec = pl.BlockSpec((1, S, 2 * DH), lambda b, hp, s: (b, 0, hp))
    q_spec = pl.BlockSpec((1, _BQ, 2 * DH), lambda b, hp, s: (b, s, hp))
    return pl.pallas_call(
        _flash_body,
        grid=grid,
        in_specs=[q_spec, kv_spec, kv_spec],
        out_specs=q_spec,
        out_shape=jax.ShapeDtypeStruct((B, S, D), BF16),
        compiler_params=pltpu.CompilerParams(
            dimension_semantics=("parallel", "parallel", "arbitrary")),
    )(q, k, v)


_PROJ_BLK = 512


def _proj_body(a_ref, w_ref, o_ref):
    o_ref[0] = lax.dot_general(a_ref[0], w_ref[...],
                               (((1,), (1,)), ((), ())),
                               preferred_element_type=F32)


def _proj_call(a, w):
    grid = (B, S // _PROJ_BLK)
    return pl.pallas_call(
        _proj_body,
        grid=grid,
        in_specs=[
            pl.BlockSpec((1, _PROJ_BLK, D), lambda b, s: (b, s, 0)),
            pl.BlockSpec((D, D), lambda b, s: (0, 0)),
        ],
        out_specs=pl.BlockSpec((1, _PROJ_BLK, D), lambda b, s: (b, s, 0)),
        out_shape=jax.ShapeDtypeStruct((B, S, D), F32),
        compiler_params=pltpu.CompilerParams(
            dimension_semantics=("parallel", "parallel")),
    )(a, w)


def kernel(x, idx_qk, idx_v, idx_q, idx_k, idx_v2,
           soft_qk, soft_v, soft_q, soft_k, soft_v2,
           feature_qk_neurons, feature_v_neurons, relational_neurons,
           value_neurons, W_O):
    table = jnp.concatenate(
        [feature_qk_neurons, feature_v_neurons, relational_neurons,
         value_neurons], axis=0)
    idx_all = jnp.concatenate(
        [idx_qk, idx_v + POOL, idx_q + 2 * POOL, idx_k + 2 * POOL,
         idx_v2 + 3 * POOL], axis=1).astype(jnp.int32)
    g = _gather_rows(table, idx_all.reshape(ROWS))
    g = g.reshape(B, NSEL, D).astype(BF16)
    q, k, v = _qkv_call(
        x, g,
        soft_qk.reshape(B, 1, TOPK), soft_v.reshape(B, 1, TOPK),
        soft_q.reshape(B, 1, TOPK), soft_k.reshape(B, 1, TOPK),
        soft_v2.reshape(B, 1, TOPK))
    attn = _flash_call(q, k, v)
    return _proj_call(attn, W_O.astype(BF16))
